# Initial kernel scaffold; baseline (speedup 1.0000x reference)
import jax
import jax.numpy as jnp
from jax.experimental import pallas as pl


def kernel(rel_src, rel_dst, rel_type, node_id, attr_pair_id, attr_dst, dynamic_emb, emb_rel, q, W_m, W_vm, W_g, b_g, W_4, b_vec, rgcn_w_neigh_1, rgcn_w_loop_1, rgcn_w_neigh_2, rgcn_w_loop_2):
    raise NotImplementedError("write your pallas kernel here")


# speedup vs baseline: 2.5570x; 2.5570x over previous
import functools

import jax
import jax.numpy as jnp
from jax import lax
from jax.experimental import pallas as pl
from jax.experimental.pallas import tpu as pltpu
from jax.experimental.pallas import tpu_sc as plsc

N = 10000
H = 128
NA = 10
NC, NS = 2, 16
NW = NC * NS

_RRELU_SLOPE = (1.0 / 8.0 + 1.0 / 3.0) / 2.0

E_CH = 128
ACC_ROWS = N + 112


def _seg_sum_rows(table, src_idx, dst_idx, zeros, n_chunks):
  mesh = plsc.VectorSubcoreMesh(core_axis_name="c", subcore_axis_name="s",
                                num_cores=NC, num_subcores=NS)

  @functools.partial(
      pl.kernel,
      out_type=jax.ShapeDtypeStruct((NC, ACC_ROWS, H), jnp.float32),
      mesh=mesh,
      scratch_types=[
          pltpu.VMEM((E_CH,), jnp.int32),
          pltpu.VMEM((E_CH,), jnp.int32),
          pltpu.VMEM((E_CH, H), jnp.float32),
          pltpu.VMEM_SHARED((ACC_ROWS, H), jnp.float32),
      ],
  )
  def k(table_h, src_h, dst_h, zeros_h, out_h, src_v, dst_v, rows_v, acc):
    c = lax.axis_index("c")
    s = lax.axis_index("s")
    wid = s * NC + c
    rpt = ACC_ROWS // NS
    pltpu.sync_copy(zeros_h.at[pl.ds(s * rpt, rpt)], acc.at[pl.ds(s * rpt, rpt)])
    plsc.subcore_barrier()

    @pl.loop(0, n_chunks)
    def _chunks(i):
      base = (wid * n_chunks + i) * E_CH
      pltpu.sync_copy(src_h.at[pl.ds(base, E_CH)], src_v)
      pltpu.sync_copy(dst_h.at[pl.ds(base, E_CH)], dst_v)
      pltpu.sync_copy(table_h.at[src_v], rows_v)
      pltpu.sync_copy(rows_v, acc.at[dst_v], add=True)

    plsc.subcore_barrier()

    @pl.when(s == 0)
    def _():
      pltpu.sync_copy(acc, out_h.at[c])

  return k(table, src_idx, dst_idx, zeros)


V_CH = 8
P_CH = V_CH * NA
V_PT = 320
N_ACH = V_PT // V_CH
V_PAD = NW * V_PT
P_PAD = V_PAD * NA


def _attr_m(p16_flat, h, dst_pad, sidx_pad):
  mesh = plsc.VectorSubcoreMesh(core_axis_name="c", subcore_axis_name="s",
                                num_cores=NC, num_subcores=NS)

  @functools.partial(
      pl.kernel,
      out_type=jax.ShapeDtypeStruct((V_PAD, H), jnp.float32),
      mesh=mesh,
      scratch_types=[
          pltpu.VMEM((P_CH,), jnp.int32),
          pltpu.VMEM((P_CH,), jnp.int32),
          pltpu.VMEM((96,), jnp.float32),
          pltpu.VMEM((P_CH, H), jnp.float32),
          pltpu.VMEM((V_CH, H), jnp.float32),
      ],
  )
  def k(p16_h, h_h, dst_h, sidx_h, m_out, dst_v, sidx_v, sbuf, rows_v, outbuf):
    c = lax.axis_index("c")
    s = lax.axis_index("s")
    wid = s * NC + c
    lane = lax.iota(jnp.int32, 16)

    @pl.loop(0, N_ACH)
    def _chunks(i):
      pbase = wid * (V_PT * NA) + i * P_CH
      pltpu.sync_copy(dst_h.at[pl.ds(pbase, P_CH)], dst_v)
      pltpu.sync_copy(sidx_h.at[pl.ds(pbase, P_CH)], sidx_v)
      pltpu.sync_copy(p16_h.at[sidx_v], sbuf.at[pl.ds(0, P_CH)])
      pltpu.sync_copy(h_h.at[dst_v], rows_v)
      for j in range(V_CH):
        s16 = sbuf[pl.ds(10 * j, 16)]
        sc = [s16[l] for l in range(NA)]
        mx = sc[0]
        for l in range(1, NA):
          mx = jnp.maximum(mx, sc[l])
        e = jnp.where(lane < NA, jnp.exp(s16 - mx), 0.0)
        es = [e[l] for l in range(NA)]
        tot = es[0]
        for l in range(1, NA):
          tot = tot + es[l]
        w_vec = e / tot
        ws = [w_vec[l] for l in range(NA)]
        accs = [jnp.zeros((16,), jnp.float32) for _ in range(H // 16)]
        for m in range(NA):
          wm = ws[m]
          for b in range(H // 16):
            accs[b] = accs[b] + wm * rows_v[NA * j + m, pl.ds(16 * b, 16)]
        for b in range(H // 16):
          outbuf[j, pl.ds(16 * b, 16)] = accs[b]
      vbase = wid * V_PT + i * V_CH
      pltpu.sync_copy(outbuf, m_out.at[pl.ds(vbase, V_CH)])

  return k(p16_flat, h, dst_pad, sidx_pad)


BR = 1000
_G = N // BR


def _dot_t(a, b_ref):
  return lax.dot_general(a, b_ref, (((1,), (1,)), ((), ())),
                         preferred_element_type=jnp.float32)


def _dot(a, b_ref):
  return lax.dot_general(a, b_ref, (((1,), (0,)), ((), ())),
                         preferred_element_type=jnp.float32)


def _prep_p16(h, emb_rel, w_m, w_vm, q):
  def f(h_ref, er_ref, wm_ref, wvm_ref, q_ref, o_ref):
    a = _dot_t(er_ref[:NA], wm_ref[...])
    hw = _dot_t(h_ref[...], wvm_ref[...])
    qt = q_ref[...].reshape(H, 1)
    cols = [_dot(jnp.tanh(hw + a[m][None, :]), qt) for m in range(NA)]
    o_ref[...] = jnp.concatenate(
        cols + [jnp.zeros((BR, 16 - NA), jnp.float32)], axis=1)

  return pl.pallas_call(
      f, grid=(_G,),
      in_specs=[
          pl.BlockSpec((BR, H), lambda i: (i, 0)),
          pl.BlockSpec(emb_rel.shape, lambda i: (0, 0)),
          pl.BlockSpec((H, H), lambda i: (0, 0)),
          pl.BlockSpec((H, H), lambda i: (0, 0)),
          pl.BlockSpec((1, H), lambda i: (0, 0)),
      ],
      out_specs=pl.BlockSpec((BR, 16), lambda i: (i, 0)),
      out_shape=jax.ShapeDtypeStruct((N, 16), jnp.float32),
  )(h, emb_rel, w_m, w_vm, q)


def _gate_u(h, m_full, wg1, wg2, b_g, w_4, b_vec):
  def f(h_ref, m_ref, wg1_ref, wg2_ref, bg_ref, w4_ref, bv_ref, va_ref, u_ref):
    hb = h_ref[...]
    mb = m_ref[...]
    g = jax.nn.sigmoid(_dot(hb, wg1_ref[...]) + _dot(mb, wg2_ref[...])
                       + bg_ref[...])
    va = (1.0 - g) * hb + g * mb
    va_ref[...] = va
    u_ref[...] = jax.nn.sigmoid(_dot(va, w4_ref[...]) + bv_ref[...])

  return pl.pallas_call(
      f, grid=(_G,),
      in_specs=[
          pl.BlockSpec((BR, H), lambda i: (i, 0)),
          pl.BlockSpec((BR, H), lambda i: (i, 0)),
          pl.BlockSpec((H, H), lambda i: (0, 0)),
          pl.BlockSpec((H, H), lambda i: (0, 0)),
          pl.BlockSpec((1, H), lambda i: (0, 0)),
          pl.BlockSpec((H, H), lambda i: (0, 0)),
          pl.BlockSpec((1, H), lambda i: (0, 0)),
      ],
      out_specs=[pl.BlockSpec((BR, H), lambda i: (i, 0)),
                 pl.BlockSpec((BR, H), lambda i: (i, 0))],
      out_shape=[jax.ShapeDtypeStruct((N, H), jnp.float32),
                 jax.ShapeDtypeStruct((N, H), jnp.float32)],
  )(h, m_full, wg1, wg2, b_g, w_4, b_vec)


def _rgcn_dense(s_par, r_par, h_in, w_n, w_l, u=None, v_attr=None):
  fuse = u is not None

  def f(*refs):
    if fuse:
      (s_ref, r_ref, h_ref, wn_ref, wl_ref, u_ref, va_ref, o_ref) = refs
    else:
      (s_ref, r_ref, h_ref, wn_ref, wl_ref, o_ref) = refs
    pre = (s_ref[0] + s_ref[1] + r_ref[0] + r_ref[1])
    x = _dot(pre, wn_ref[...]) + _dot(h_ref[...], wl_ref[...])
    hh = jnp.where(x >= 0, x, x * _RRELU_SLOPE)
    if fuse:
      ub = u_ref[...]
      o_ref[...] = ub * hh + (1.0 - ub) * va_ref[...]
    else:
      o_ref[...] = hh

  spec3 = pl.BlockSpec((NC, BR, H), lambda i: (0, i, 0))
  specb = pl.BlockSpec((BR, H), lambda i: (i, 0))
  specw = pl.BlockSpec((H, H), lambda i: (0, 0))
  in_specs = [spec3, spec3, specb, specw, specw]
  args = [s_par, r_par, h_in, w_n, w_l]
  if fuse:
    in_specs += [specb, specb]
    args += [u, v_attr]
  return pl.pallas_call(
      f, grid=(_G,), in_specs=in_specs,
      out_specs=specb,
      out_shape=jax.ShapeDtypeStruct((N, H), jnp.float32),
  )(*args)


def kernel(rel_src, rel_dst, rel_type, node_id, attr_pair_id, attr_dst,
           dynamic_emb, emb_rel, q, W_m, W_vm, W_g, b_g, W_4, b_vec,
           rgcn_w_neigh_1, rgcn_w_loop_1, rgcn_w_neigh_2, rgcn_w_loop_2):
  h = dynamic_emb
  e = rel_src.shape[0]
  n_chunks = -(-e // (NW * E_CH))
  e_pad = NW * E_CH * n_chunks

  def pad1(x, total, fill):
    return jnp.concatenate(
        [x.astype(jnp.int32), jnp.full((total - x.shape[0],), fill, jnp.int32)])

  src_p = pad1(rel_src, e_pad, 0)
  dst_p = pad1(rel_dst, e_pad, N)
  typ_p = pad1(rel_type, e_pad, 0)
  adst_p = pad1(attr_dst, P_PAD, 0)
  sidx = attr_dst.astype(jnp.int32) * 16 + (attr_pair_id % NA).astype(jnp.int32)
  sidx_p = pad1(sidx, P_PAD, 0)
  zeros = jnp.zeros((ACC_ROWS, H), jnp.float32)

  r_par = _seg_sum_rows(emb_rel, typ_p, dst_p, zeros, n_chunks)
  s1_par = _seg_sum_rows(h, src_p, dst_p, zeros, n_chunks)

  p16 = _prep_p16(h, emb_rel, W_m, W_vm, q)
  m_full = _attr_m(p16.reshape(-1), h, adst_p, sidx_p)
  v_attr, u_gate = _gate_u(h, m_full, W_g[:H], W_g[H:], b_g, W_4, b_vec)

  hh1 = _rgcn_dense(s1_par, r_par, h, rgcn_w_neigh_1, rgcn_w_loop_1)
  s2_par = _seg_sum_rows(hh1, src_p, dst_p, zeros, n_chunks)
  h_new = _rgcn_dense(s2_par, r_par, hh1, rgcn_w_neigh_2, rgcn_w_loop_2,
                      u=u_gate, v_attr=v_attr)
  return (h_new, emb_rel)


# speedup vs baseline: 2.9618x; 1.1583x over previous
import functools

import jax
import jax.numpy as jnp
from jax import lax
from jax.experimental import pallas as pl
from jax.experimental.pallas import tpu as pltpu
from jax.experimental.pallas import tpu_sc as plsc

N = 10000
H = 128
NA = 10
NC, NS = 2, 16
NW = NC * NS

_RRELU_SLOPE = (1.0 / 8.0 + 1.0 / 3.0) / 2.0

E_CH = 128
ACC_ROWS = N + 112


def _seg_sum_rows(table, src_idx, dst_idx, zeros, n_chunks):
  mesh = plsc.VectorSubcoreMesh(core_axis_name="c", subcore_axis_name="s",
                                num_cores=NC, num_subcores=NS)

  @functools.partial(
      pl.kernel,
      out_type=jax.ShapeDtypeStruct((NC, ACC_ROWS, H), jnp.float32),
      mesh=mesh,
      scratch_types=[
          pltpu.VMEM((n_chunks, E_CH), jnp.int32),
          pltpu.VMEM((n_chunks, E_CH), jnp.int32),
          pltpu.VMEM((2, E_CH, H), jnp.float32),
          pltpu.VMEM_SHARED((ACC_ROWS, H), jnp.float32),
          pltpu.SemaphoreType.DMA,
          pltpu.SemaphoreType.DMA,
      ],
  )
  def k(table_h, src_h, dst_h, zeros_h, out_h, src2d, dst2d, rows, acc,
        sem0, sem1):
    c = lax.axis_index("c")
    s = lax.axis_index("s")
    wid = s * NC + c
    rpt = ACC_ROWS // NS
    sems = (sem0, sem1)
    pltpu.sync_copy(zeros_h.at[pl.ds(s * rpt, rpt)], acc.at[pl.ds(s * rpt, rpt)])
    pltpu.sync_copy(src_h.at[wid], src2d)
    pltpu.sync_copy(dst_h.at[wid], dst2d)
    plsc.subcore_barrier()

    def start_gather(ic, b):
      pltpu.async_copy(table_h.at[src2d.at[ic]], rows.at[b], sems[b])

    def wait_gather(b):
      pltpu.make_async_copy(table_h.at[src2d.at[0]], rows.at[b], sems[b]).wait()

    start_gather(0, 0)

    @pl.loop(0, n_chunks // 2)
    def _steps(i):
      for b in range(2):
        ic = 2 * i + b
        wait_gather(b)

        @pl.when(ic + 1 < n_chunks)
        def _():
          start_gather(ic + 1, 1 - b)

        pltpu.sync_copy(rows.at[b], acc.at[dst2d.at[ic]], add=True)

    plsc.subcore_barrier()

    @pl.when(s == 0)
    def _():
      pltpu.sync_copy(acc, out_h.at[c])

  return k(table, src_idx, dst_idx, zeros)


V_CH = 8
P_CH = V_CH * NA
V_PT = 320
N_ACH = V_PT // V_CH
V_PAD = NW * V_PT
P_PAD = V_PAD * NA


def _attr_m(p16_flat, h, dst_pad, sidx_pad):
  mesh = plsc.VectorSubcoreMesh(core_axis_name="c", subcore_axis_name="s",
                                num_cores=NC, num_subcores=NS)

  @functools.partial(
      pl.kernel,
      out_type=jax.ShapeDtypeStruct((V_PAD, H), jnp.float32),
      mesh=mesh,
      scratch_types=[
          pltpu.VMEM((P_CH,), jnp.int32),
          pltpu.VMEM((P_CH,), jnp.int32),
          pltpu.VMEM((96,), jnp.float32),
          pltpu.VMEM((P_CH, H), jnp.float32),
          pltpu.VMEM((V_CH, H), jnp.float32),
      ],
  )
  def k(p16_h, h_h, dst_h, sidx_h, m_out, dst_v, sidx_v, sbuf, rows_v, outbuf):
    c = lax.axis_index("c")
    s = lax.axis_index("s")
    wid = s * NC + c
    lane = lax.iota(jnp.int32, 16)

    @pl.loop(0, N_ACH)
    def _chunks(i):
      pbase = wid * (V_PT * NA) + i * P_CH
      pltpu.sync_copy(dst_h.at[pl.ds(pbase, P_CH)], dst_v)
      pltpu.sync_copy(sidx_h.at[pl.ds(pbase, P_CH)], sidx_v)
      pltpu.sync_copy(p16_h.at[sidx_v], sbuf.at[pl.ds(0, P_CH)])
      pltpu.sync_copy(h_h.at[dst_v], rows_v)
      for j in range(V_CH):
        s16 = sbuf[pl.ds(10 * j, 16)]
        sc = [s16[l] for l in range(NA)]
        mx = sc[0]
        for l in range(1, NA):
          mx = jnp.maximum(mx, sc[l])
        e = jnp.where(lane < NA, jnp.exp(s16 - mx), 0.0)
        es = [e[l] for l in range(NA)]
        tot = es[0]
        for l in range(1, NA):
          tot = tot + es[l]
        w_vec = e / tot
        ws = [w_vec[l] for l in range(NA)]
        accs = [jnp.zeros((16,), jnp.float32) for _ in range(H // 16)]
        for m in range(NA):
          wm = ws[m]
          for b in range(H // 16):
            accs[b] = accs[b] + wm * rows_v[NA * j + m, pl.ds(16 * b, 16)]
        for b in range(H // 16):
          outbuf[j, pl.ds(16 * b, 16)] = accs[b]
      vbase = wid * V_PT + i * V_CH
      pltpu.sync_copy(outbuf, m_out.at[pl.ds(vbase, V_CH)])

  return k(p16_flat, h, dst_pad, sidx_pad)


BR = 1000
_G = N // BR


def _dot_t(a, b_ref):
  return lax.dot_general(a, b_ref, (((1,), (1,)), ((), ())),
                         preferred_element_type=jnp.float32)


def _dot(a, b_ref):
  return lax.dot_general(a, b_ref, (((1,), (0,)), ((), ())),
                         preferred_element_type=jnp.float32)


def _prep_p16(h, emb_rel, w_m, w_vm, q):
  def f(h_ref, er_ref, wm_ref, wvm_ref, q_ref, o_ref):
    a = _dot_t(er_ref[:NA], wm_ref[...])
    hw = _dot_t(h_ref[...], wvm_ref[...])
    qt = q_ref[...].reshape(H, 1)
    cols = [_dot(jnp.tanh(hw + a[m][None, :]), qt) for m in range(NA)]
    o_ref[...] = jnp.concatenate(
        cols + [jnp.zeros((BR, 16 - NA), jnp.float32)], axis=1)

  return pl.pallas_call(
      f, grid=(_G,),
      in_specs=[
          pl.BlockSpec((BR, H), lambda i: (i, 0)),
          pl.BlockSpec(emb_rel.shape, lambda i: (0, 0)),
          pl.BlockSpec((H, H), lambda i: (0, 0)),
          pl.BlockSpec((H, H), lambda i: (0, 0)),
          pl.BlockSpec((1, H), lambda i: (0, 0)),
      ],
      out_specs=pl.BlockSpec((BR, 16), lambda i: (i, 0)),
      out_shape=jax.ShapeDtypeStruct((N, 16), jnp.float32),
  )(h, emb_rel, w_m, w_vm, q)


def _gate_u(h, m_full, wg1, wg2, b_g, w_4, b_vec):
  def f(h_ref, m_ref, wg1_ref, wg2_ref, bg_ref, w4_ref, bv_ref, va_ref, u_ref):
    hb = h_ref[...]
    mb = m_ref[...]
    g = jax.nn.sigmoid(_dot(hb, wg1_ref[...]) + _dot(mb, wg2_ref[...])
                       + bg_ref[...])
    va = (1.0 - g) * hb + g * mb
    va_ref[...] = va
    u_ref[...] = jax.nn.sigmoid(_dot(va, w4_ref[...]) + bv_ref[...])

  return pl.pallas_call(
      f, grid=(_G,),
      in_specs=[
          pl.BlockSpec((BR, H), lambda i: (i, 0)),
          pl.BlockSpec((BR, H), lambda i: (i, 0)),
          pl.BlockSpec((H, H), lambda i: (0, 0)),
          pl.BlockSpec((H, H), lambda i: (0, 0)),
          pl.BlockSpec((1, H), lambda i: (0, 0)),
          pl.BlockSpec((H, H), lambda i: (0, 0)),
          pl.BlockSpec((1, H), lambda i: (0, 0)),
      ],
      out_specs=[pl.BlockSpec((BR, H), lambda i: (i, 0)),
                 pl.BlockSpec((BR, H), lambda i: (i, 0))],
      out_shape=[jax.ShapeDtypeStruct((N, H), jnp.float32),
                 jax.ShapeDtypeStruct((N, H), jnp.float32)],
  )(h, m_full, wg1, wg2, b_g, w_4, b_vec)


def _rgcn_dense(s_par, r_par, h_in, w_n, w_l, u=None, v_attr=None):
  fuse = u is not None

  def f(*refs):
    if fuse:
      (s_ref, r_ref, h_ref, wn_ref, wl_ref, u_ref, va_ref, o_ref) = refs
    else:
      (s_ref, r_ref, h_ref, wn_ref, wl_ref, o_ref) = refs
    pre = (s_ref[0] + s_ref[1] + r_ref[0] + r_ref[1])
    x = _dot(pre, wn_ref[...]) + _dot(h_ref[...], wl_ref[...])
    hh = jnp.where(x >= 0, x, x * _RRELU_SLOPE)
    if fuse:
      ub = u_ref[...]
      o_ref[...] = ub * hh + (1.0 - ub) * va_ref[...]
    else:
      o_ref[...] = hh

  spec3 = pl.BlockSpec((NC, BR, H), lambda i: (0, i, 0))
  specb = pl.BlockSpec((BR, H), lambda i: (i, 0))
  specw = pl.BlockSpec((H, H), lambda i: (0, 0))
  in_specs = [spec3, spec3, specb, specw, specw]
  args = [s_par, r_par, h_in, w_n, w_l]
  if fuse:
    in_specs += [specb, specb]
    args += [u, v_attr]
  return pl.pallas_call(
      f, grid=(_G,), in_specs=in_specs,
      out_specs=specb,
      out_shape=jax.ShapeDtypeStruct((N, H), jnp.float32),
  )(*args)


def kernel(rel_src, rel_dst, rel_type, node_id, attr_pair_id, attr_dst,
           dynamic_emb, emb_rel, q, W_m, W_vm, W_g, b_g, W_4, b_vec,
           rgcn_w_neigh_1, rgcn_w_loop_1, rgcn_w_neigh_2, rgcn_w_loop_2):
  h = dynamic_emb
  e = rel_src.shape[0]
  n_chunks = -(-e // (NW * E_CH))
  e_pad = NW * E_CH * n_chunks

  def pad1(x, total, fill):
    return jnp.concatenate(
        [x.astype(jnp.int32), jnp.full((total - x.shape[0],), fill, jnp.int32)])

  src_p = pad1(rel_src, e_pad, 0).reshape(NW, n_chunks, E_CH)
  dst_p = pad1(rel_dst, e_pad, N).reshape(NW, n_chunks, E_CH)
  typ_p = pad1(rel_type, e_pad, 0).reshape(NW, n_chunks, E_CH)
  adst_p = pad1(attr_dst, P_PAD, 0)
  sidx = attr_dst.astype(jnp.int32) * 16 + (attr_pair_id % NA).astype(jnp.int32)
  sidx_p = pad1(sidx, P_PAD, 0)
  zeros = jnp.zeros((ACC_ROWS, H), jnp.float32)

  r_par = _seg_sum_rows(emb_rel, typ_p, dst_p, zeros, n_chunks)
  s1_par = _seg_sum_rows(h, src_p, dst_p, zeros, n_chunks)

  p16 = _prep_p16(h, emb_rel, W_m, W_vm, q)
  m_full = _attr_m(p16.reshape(-1), h, adst_p, sidx_p)
  v_attr, u_gate = _gate_u(h, m_full, W_g[:H], W_g[H:], b_g, W_4, b_vec)

  hh1 = _rgcn_dense(s1_par, r_par, h, rgcn_w_neigh_1, rgcn_w_loop_1)
  s2_par = _seg_sum_rows(hh1, src_p, dst_p, zeros, n_chunks)
  h_new = _rgcn_dense(s2_par, r_par, hh1, rgcn_w_neigh_2, rgcn_w_loop_2,
                      u=u_gate, v_attr=v_attr)
  return (h_new, emb_rel)


# speedup vs baseline: 3.0463x; 1.0286x over previous
import functools

import jax
import jax.numpy as jnp
from jax import lax
from jax.experimental import pallas as pl
from jax.experimental.pallas import tpu as pltpu
from jax.experimental.pallas import tpu_sc as plsc

N = 10000
H = 128
NA = 10
NC, NS = 2, 16
NW = NC * NS

_RRELU_SLOPE = (1.0 / 8.0 + 1.0 / 3.0) / 2.0

E_CH = 128
ACC_ROWS = N + 112


def _seg_sum_rows(table, src_idx, dst_idx, zeros, n_chunks):
  mesh = plsc.VectorSubcoreMesh(core_axis_name="c", subcore_axis_name="s",
                                num_cores=NC, num_subcores=NS)

  @functools.partial(
      pl.kernel,
      out_type=jax.ShapeDtypeStruct((NC, ACC_ROWS, H), jnp.float32),
      mesh=mesh,
      scratch_types=[
          pltpu.VMEM((n_chunks, E_CH), jnp.int32),
          pltpu.VMEM((n_chunks, E_CH), jnp.int32),
          pltpu.VMEM((2, E_CH, H), jnp.float32),
          pltpu.VMEM_SHARED((ACC_ROWS, H), jnp.float32),
          pltpu.SemaphoreType.DMA,
          pltpu.SemaphoreType.DMA,
      ],
  )
  def k(table_h, src_h, dst_h, zeros_h, out_h, src2d, dst2d, rows, acc,
        sem0, sem1):
    c = lax.axis_index("c")
    s = lax.axis_index("s")
    wid = s * NC + c
    rpt = ACC_ROWS // NS
    sems = (sem0, sem1)
    pltpu.sync_copy(zeros_h.at[pl.ds(s * rpt, rpt)], acc.at[pl.ds(s * rpt, rpt)])
    pltpu.sync_copy(src_h.at[wid], src2d)
    pltpu.sync_copy(dst_h.at[wid], dst2d)
    plsc.subcore_barrier()

    def start_gather(ic, b):
      pltpu.async_copy(table_h.at[src2d.at[ic]], rows.at[b], sems[b])

    def wait_gather(b):
      pltpu.make_async_copy(table_h.at[src2d.at[0]], rows.at[b], sems[b]).wait()

    start_gather(0, 0)

    @pl.loop(0, n_chunks // 2)
    def _steps(i):
      for b in range(2):
        ic = 2 * i + b
        wait_gather(b)

        @pl.when(ic + 1 < n_chunks)
        def _():
          start_gather(ic + 1, 1 - b)

        pltpu.sync_copy(rows.at[b], acc.at[dst2d.at[ic]], add=True)

    plsc.subcore_barrier()

    @pl.when(s == 0)
    def _():
      pltpu.sync_copy(acc, out_h.at[c])

  return k(table, src_idx, dst_idx, zeros)


V_CH = 8
P_CH = V_CH * NA
V_PT = 320
N_ACH = V_PT // V_CH
V_PAD = NW * V_PT
P_PAD = V_PAD * NA


def _attr_m(p16_flat, h, dst_pad, sidx_pad):
  mesh = plsc.VectorSubcoreMesh(core_axis_name="c", subcore_axis_name="s",
                                num_cores=NC, num_subcores=NS)

  @functools.partial(
      pl.kernel,
      out_type=jax.ShapeDtypeStruct((V_PAD, H), jnp.float32),
      mesh=mesh,
      scratch_types=[
          pltpu.VMEM((N_ACH * P_CH,), jnp.int32),
          pltpu.VMEM((N_ACH * P_CH,), jnp.int32),
          pltpu.VMEM((2, 96), jnp.float32),
          pltpu.VMEM((2, P_CH, H), jnp.float32),
          pltpu.VMEM((V_CH, H), jnp.float32),
          pltpu.SemaphoreType.DMA,
          pltpu.SemaphoreType.DMA,
          pltpu.SemaphoreType.DMA,
          pltpu.SemaphoreType.DMA,
      ],
  )
  def k(p16_h, h_h, dst_h, sidx_h, m_out, dst_all, sidx_all, sbuf, rows,
        outbuf, ss0, ss1, rs0, rs1):
    c = lax.axis_index("c")
    s = lax.axis_index("s")
    wid = s * NC + c
    lane = lax.iota(jnp.int32, 16)
    ssems = (ss0, ss1)
    rsems = (rs0, rs1)
    pltpu.sync_copy(dst_h.at[pl.ds(wid * (V_PT * NA), N_ACH * P_CH)], dst_all)
    pltpu.sync_copy(sidx_h.at[pl.ds(wid * (V_PT * NA), N_ACH * P_CH)], sidx_all)

    def start_gathers(ic, b):
      pltpu.async_copy(p16_h.at[sidx_all.at[pl.ds(ic * P_CH, P_CH)]],
                       sbuf.at[b, pl.ds(0, P_CH)], ssems[b])
      pltpu.async_copy(h_h.at[dst_all.at[pl.ds(ic * P_CH, P_CH)]],
                       rows.at[b], rsems[b])

    def wait_gathers(b):
      pltpu.make_async_copy(p16_h.at[sidx_all.at[pl.ds(0, P_CH)]],
                            sbuf.at[b, pl.ds(0, P_CH)], ssems[b]).wait()
      pltpu.make_async_copy(h_h.at[dst_all.at[pl.ds(0, P_CH)]],
                            rows.at[b], rsems[b]).wait()

    start_gathers(0, 0)

    @pl.loop(0, N_ACH // 2)
    def _steps(i):
     for pb in range(2):
      ic = 2 * i + pb
      wait_gathers(pb)

      @pl.when(ic + 1 < N_ACH)
      def _():
        start_gathers(ic + 1, 1 - pb)

      for j in range(V_CH):
        s16 = sbuf[pb, pl.ds(10 * j, 16)]
        sc = [s16[l] for l in range(NA)]
        mx = sc[0]
        for l in range(1, NA):
          mx = jnp.maximum(mx, sc[l])
        e = jnp.where(lane < NA, jnp.exp(s16 - mx), 0.0)
        es = [e[l] for l in range(NA)]
        tot = es[0]
        for l in range(1, NA):
          tot = tot + es[l]
        w_vec = e / tot
        ws = [w_vec[l] for l in range(NA)]
        accs = [jnp.zeros((16,), jnp.float32) for _ in range(H // 16)]
        for m in range(NA):
          wm = ws[m]
          for b in range(H // 16):
            accs[b] = accs[b] + wm * rows[pb, NA * j + m, pl.ds(16 * b, 16)]
        for b in range(H // 16):
          outbuf[j, pl.ds(16 * b, 16)] = accs[b]
      vbase = wid * V_PT + ic * V_CH
      pltpu.sync_copy(outbuf, m_out.at[pl.ds(vbase, V_CH)])

  return k(p16_flat, h, dst_pad, sidx_pad)


BR = 1000
_G = N // BR


def _dot_t(a, b_ref):
  return lax.dot_general(a, b_ref, (((1,), (1,)), ((), ())),
                         preferred_element_type=jnp.float32)


def _dot(a, b_ref):
  return lax.dot_general(a, b_ref, (((1,), (0,)), ((), ())),
                         preferred_element_type=jnp.float32)


def _prep_p16(h, emb_rel, w_m, w_vm, q):
  def f(h_ref, er_ref, wm_ref, wvm_ref, q_ref, o_ref):
    a = _dot_t(er_ref[:NA], wm_ref[...])
    hw = _dot_t(h_ref[...], wvm_ref[...])
    qt = q_ref[...].reshape(H, 1)
    cols = [_dot(jnp.tanh(hw + a[m][None, :]), qt) for m in range(NA)]
    o_ref[...] = jnp.concatenate(
        cols + [jnp.zeros((BR, 16 - NA), jnp.float32)], axis=1)

  return pl.pallas_call(
      f, grid=(_G,),
      in_specs=[
          pl.BlockSpec((BR, H), lambda i: (i, 0)),
          pl.BlockSpec(emb_rel.shape, lambda i: (0, 0)),
          pl.BlockSpec((H, H), lambda i: (0, 0)),
          pl.BlockSpec((H, H), lambda i: (0, 0)),
          pl.BlockSpec((1, H), lambda i: (0, 0)),
      ],
      out_specs=pl.BlockSpec((BR, 16), lambda i: (i, 0)),
      out_shape=jax.ShapeDtypeStruct((N, 16), jnp.float32),
  )(h, emb_rel, w_m, w_vm, q)


def _gate_u(h, m_full, wg1, wg2, b_g, w_4, b_vec):
  def f(h_ref, m_ref, wg1_ref, wg2_ref, bg_ref, w4_ref, bv_ref, va_ref, u_ref):
    hb = h_ref[...]
    mb = m_ref[...]
    g = jax.nn.sigmoid(_dot(hb, wg1_ref[...]) + _dot(mb, wg2_ref[...])
                       + bg_ref[...])
    va = (1.0 - g) * hb + g * mb
    va_ref[...] = va
    u_ref[...] = jax.nn.sigmoid(_dot(va, w4_ref[...]) + bv_ref[...])

  return pl.pallas_call(
      f, grid=(_G,),
      in_specs=[
          pl.BlockSpec((BR, H), lambda i: (i, 0)),
          pl.BlockSpec((BR, H), lambda i: (i, 0)),
          pl.BlockSpec((H, H), lambda i: (0, 0)),
          pl.BlockSpec((H, H), lambda i: (0, 0)),
          pl.BlockSpec((1, H), lambda i: (0, 0)),
          pl.BlockSpec((H, H), lambda i: (0, 0)),
          pl.BlockSpec((1, H), lambda i: (0, 0)),
      ],
      out_specs=[pl.BlockSpec((BR, H), lambda i: (i, 0)),
                 pl.BlockSpec((BR, H), lambda i: (i, 0))],
      out_shape=[jax.ShapeDtypeStruct((N, H), jnp.float32),
                 jax.ShapeDtypeStruct((N, H), jnp.float32)],
  )(h, m_full, wg1, wg2, b_g, w_4, b_vec)


def _rgcn_dense(s_par, r_par, h_in, w_n, w_l, u=None, v_attr=None):
  fuse = u is not None

  def f(*refs):
    if fuse:
      (s_ref, r_ref, h_ref, wn_ref, wl_ref, u_ref, va_ref, o_ref) = refs
    else:
      (s_ref, r_ref, h_ref, wn_ref, wl_ref, o_ref) = refs
    pre = (s_ref[0] + s_ref[1] + r_ref[0] + r_ref[1])
    x = _dot(pre, wn_ref[...]) + _dot(h_ref[...], wl_ref[...])
    hh = jnp.where(x >= 0, x, x * _RRELU_SLOPE)
    if fuse:
      ub = u_ref[...]
      o_ref[...] = ub * hh + (1.0 - ub) * va_ref[...]
    else:
      o_ref[...] = hh

  spec3 = pl.BlockSpec((NC, BR, H), lambda i: (0, i, 0))
  specb = pl.BlockSpec((BR, H), lambda i: (i, 0))
  specw = pl.BlockSpec((H, H), lambda i: (0, 0))
  in_specs = [spec3, spec3, specb, specw, specw]
  args = [s_par, r_par, h_in, w_n, w_l]
  if fuse:
    in_specs += [specb, specb]
    args += [u, v_attr]
  return pl.pallas_call(
      f, grid=(_G,), in_specs=in_specs,
      out_specs=specb,
      out_shape=jax.ShapeDtypeStruct((N, H), jnp.float32),
  )(*args)


def kernel(rel_src, rel_dst, rel_type, node_id, attr_pair_id, attr_dst,
           dynamic_emb, emb_rel, q, W_m, W_vm, W_g, b_g, W_4, b_vec,
           rgcn_w_neigh_1, rgcn_w_loop_1, rgcn_w_neigh_2, rgcn_w_loop_2):
  h = dynamic_emb
  e = rel_src.shape[0]
  n_chunks = -(-e // (NW * E_CH))
  e_pad = NW * E_CH * n_chunks

  def pad1(x, total, fill):
    return jnp.concatenate(
        [x.astype(jnp.int32), jnp.full((total - x.shape[0],), fill, jnp.int32)])

  src_p = pad1(rel_src, e_pad, 0).reshape(NW, n_chunks, E_CH)
  dst_p = pad1(rel_dst, e_pad, N).reshape(NW, n_chunks, E_CH)
  typ_p = pad1(rel_type, e_pad, 0).reshape(NW, n_chunks, E_CH)
  adst_p = pad1(attr_dst, P_PAD, 0)
  sidx = attr_dst.astype(jnp.int32) * 16 + (attr_pair_id % NA).astype(jnp.int32)
  sidx_p = pad1(sidx, P_PAD, 0)
  zeros = jnp.zeros((ACC_ROWS, H), jnp.float32)

  r_par = _seg_sum_rows(emb_rel, typ_p, dst_p, zeros, n_chunks)
  s1_par = _seg_sum_rows(h, src_p, dst_p, zeros, n_chunks)

  p16 = _prep_p16(h, emb_rel, W_m, W_vm, q)
  m_full = _attr_m(p16.reshape(-1), h, adst_p, sidx_p)
  v_attr, u_gate = _gate_u(h, m_full, W_g[:H], W_g[H:], b_g, W_4, b_vec)

  hh1 = _rgcn_dense(s1_par, r_par, h, rgcn_w_neigh_1, rgcn_w_loop_1)
  s2_par = _seg_sum_rows(hh1, src_p, dst_p, zeros, n_chunks)
  h_new = _rgcn_dense(s2_par, r_par, hh1, rgcn_w_neigh_2, rgcn_w_loop_2,
                      u=u_gate, v_attr=v_attr)
  return (h_new, emb_rel)


# speedup vs baseline: 3.1531x; 1.0350x over previous
import functools

import jax
import jax.numpy as jnp
from jax import lax
from jax.experimental import pallas as pl
from jax.experimental.pallas import tpu as pltpu
from jax.experimental.pallas import tpu_sc as plsc

N = 10000
H = 128
NA = 10
NC, NS = 2, 16
NW = NC * NS

_RRELU_SLOPE = (1.0 / 8.0 + 1.0 / 3.0) / 2.0

E_CH = 128
ACC_ROWS = N + 112


def _seg_sum_rows(table, src_idx, dst_idx, zeros, n_chunks):
  mesh = plsc.VectorSubcoreMesh(core_axis_name="c", subcore_axis_name="s",
                                num_cores=NC, num_subcores=NS)

  @functools.partial(
      pl.kernel,
      out_type=jax.ShapeDtypeStruct((NC, ACC_ROWS, H), jnp.float32),
      mesh=mesh,
      scratch_types=[
          pltpu.VMEM((n_chunks, E_CH), jnp.int32),
          pltpu.VMEM((n_chunks, E_CH), jnp.int32),
          pltpu.VMEM((2, E_CH, H), jnp.float32),
          pltpu.VMEM_SHARED((ACC_ROWS, H), jnp.float32),
          pltpu.SemaphoreType.DMA,
          pltpu.SemaphoreType.DMA,
      ],
  )
  def k(table_h, src_h, dst_h, zeros_h, out_h, src2d, dst2d, rows, acc,
        sem0, sem1):
    c = lax.axis_index("c")
    s = lax.axis_index("s")
    wid = s * NC + c
    rpt = ACC_ROWS // NS
    sems = (sem0, sem1)
    pltpu.sync_copy(zeros_h.at[pl.ds(s * rpt, rpt)], acc.at[pl.ds(s * rpt, rpt)])
    pltpu.sync_copy(src_h.at[wid], src2d)
    pltpu.sync_copy(dst_h.at[wid], dst2d)
    plsc.subcore_barrier()

    def start_gather(ic, b):
      pltpu.async_copy(table_h.at[src2d.at[ic]], rows.at[b], sems[b])

    def wait_gather(b):
      pltpu.make_async_copy(table_h.at[src2d.at[0]], rows.at[b], sems[b]).wait()

    start_gather(0, 0)

    @pl.loop(0, n_chunks // 2)
    def _steps(i):
      for b in range(2):
        ic = 2 * i + b
        wait_gather(b)

        @pl.when(ic + 1 < n_chunks)
        def _():
          start_gather(ic + 1, 1 - b)

        pltpu.sync_copy(rows.at[b], acc.at[dst2d.at[ic]], add=True)

    plsc.subcore_barrier()

    @pl.when(s == 0)
    def _():
      pltpu.sync_copy(acc, out_h.at[c])

  return k(table, src_idx, dst_idx, zeros)


V_CH = 8
P_CH = V_CH * NA
V_PT = 320
N_ACH = V_PT // V_CH
V_PAD = NW * V_PT
P_PAD = V_PAD * NA


def _attr_m(p16_flat, h, dst_pad, sidx_pad):
  mesh = plsc.VectorSubcoreMesh(core_axis_name="c", subcore_axis_name="s",
                                num_cores=NC, num_subcores=NS)

  @functools.partial(
      pl.kernel,
      out_type=jax.ShapeDtypeStruct((V_PAD, H), jnp.float32),
      mesh=mesh,
      scratch_types=[
          pltpu.VMEM((N_ACH * P_CH,), jnp.int32),
          pltpu.VMEM((N_ACH * P_CH,), jnp.int32),
          pltpu.VMEM((2, 96), jnp.float32),
          pltpu.VMEM((2, P_CH, H), jnp.float32),
          pltpu.VMEM((V_CH, H), jnp.float32),
          pltpu.SemaphoreType.DMA,
          pltpu.SemaphoreType.DMA,
          pltpu.SemaphoreType.DMA,
          pltpu.SemaphoreType.DMA,
      ],
  )
  def k(p16_h, h_h, dst_h, sidx_h, m_out, dst_all, sidx_all, sbuf, rows,
        outbuf, ss0, ss1, rs0, rs1):
    c = lax.axis_index("c")
    s = lax.axis_index("s")
    wid = s * NC + c
    lane = lax.iota(jnp.int32, 16)
    ssems = (ss0, ss1)
    rsems = (rs0, rs1)
    pltpu.sync_copy(dst_h.at[pl.ds(wid * (V_PT * NA), N_ACH * P_CH)], dst_all)
    pltpu.sync_copy(sidx_h.at[pl.ds(wid * (V_PT * NA), N_ACH * P_CH)], sidx_all)

    def start_gathers(ic, b):
      pltpu.async_copy(p16_h.at[sidx_all.at[pl.ds(ic * P_CH, P_CH)]],
                       sbuf.at[b, pl.ds(0, P_CH)], ssems[b])
      pltpu.async_copy(h_h.at[dst_all.at[pl.ds(ic * P_CH, P_CH)]],
                       rows.at[b], rsems[b])

    def wait_gathers(b):
      pltpu.make_async_copy(p16_h.at[sidx_all.at[pl.ds(0, P_CH)]],
                            sbuf.at[b, pl.ds(0, P_CH)], ssems[b]).wait()
      pltpu.make_async_copy(h_h.at[dst_all.at[pl.ds(0, P_CH)]],
                            rows.at[b], rsems[b]).wait()

    start_gathers(0, 0)

    @pl.loop(0, N_ACH // 2)
    def _steps(i):
     for pb in range(2):
      ic = 2 * i + pb
      wait_gathers(pb)

      @pl.when(ic + 1 < N_ACH)
      def _():
        start_gathers(ic + 1, 1 - pb)

      for j in range(V_CH):
        s16 = sbuf[pb, pl.ds(10 * j, 16)]
        sc = [s16[l] for l in range(NA)]
        mx = sc[0]
        for l in range(1, NA):
          mx = jnp.maximum(mx, sc[l])
        e = jnp.where(lane < NA, jnp.exp(s16 - mx), 0.0)
        es = [e[l] for l in range(NA)]
        tot = es[0]
        for l in range(1, NA):
          tot = tot + es[l]
        w_vec = e / tot
        ws = [w_vec[l] for l in range(NA)]
        accs = [jnp.zeros((16,), jnp.float32) for _ in range(H // 16)]
        for m in range(NA):
          wm = ws[m]
          for b in range(H // 16):
            accs[b] = accs[b] + wm * rows[pb, NA * j + m, pl.ds(16 * b, 16)]
        for b in range(H // 16):
          outbuf[j, pl.ds(16 * b, 16)] = accs[b]
      vbase = wid * V_PT + ic * V_CH
      pltpu.sync_copy(outbuf, m_out.at[pl.ds(vbase, V_CH)])

  return k(p16_flat, h, dst_pad, sidx_pad)


BR = 1000
_G = N // BR


def _dot_t(a, b_ref):
  return lax.dot_general(a, b_ref, (((1,), (1,)), ((), ())),
                         preferred_element_type=jnp.float32)


def _dot(a, b_ref):
  return lax.dot_general(a, b_ref, (((1,), (0,)), ((), ())),
                         preferred_element_type=jnp.float32)


def _prep_p16(h, emb_rel, w_m, w_vm, q):
  def f(h_ref, er_ref, wm_ref, wvm_ref, q_ref, o_ref):
    a = _dot_t(er_ref[:NA], wm_ref[...])
    hw = _dot_t(h_ref[...], wvm_ref[...])
    qt = q_ref[...].reshape(H, 1)
    cols = [_dot(jnp.tanh(hw + a[m][None, :]), qt) for m in range(NA)]
    o_ref[...] = jnp.concatenate(
        cols + [jnp.zeros((BR, 16 - NA), jnp.float32)], axis=1)

  return pl.pallas_call(
      f, grid=(_G,),
      in_specs=[
          pl.BlockSpec((BR, H), lambda i: (i, 0)),
          pl.BlockSpec(emb_rel.shape, lambda i: (0, 0)),
          pl.BlockSpec((H, H), lambda i: (0, 0)),
          pl.BlockSpec((H, H), lambda i: (0, 0)),
          pl.BlockSpec((1, H), lambda i: (0, 0)),
      ],
      out_specs=pl.BlockSpec((BR, 16), lambda i: (i, 0)),
      out_shape=jax.ShapeDtypeStruct((N, 16), jnp.float32),
  )(h, emb_rel, w_m, w_vm, q)


def _gate_u(h, m_full, wg1, wg2, b_g, w_4, b_vec):
  def f(h_ref, m_ref, wg1_ref, wg2_ref, bg_ref, w4_ref, bv_ref, va_ref, u_ref):
    hb = h_ref[...]
    mb = m_ref[...]
    g = jax.nn.sigmoid(_dot(hb, wg1_ref[...]) + _dot(mb, wg2_ref[...])
                       + bg_ref[...])
    va = (1.0 - g) * hb + g * mb
    va_ref[...] = va
    u_ref[...] = jax.nn.sigmoid(_dot(va, w4_ref[...]) + bv_ref[...])

  return pl.pallas_call(
      f, grid=(_G,),
      in_specs=[
          pl.BlockSpec((BR, H), lambda i: (i, 0)),
          pl.BlockSpec((BR, H), lambda i: (i, 0)),
          pl.BlockSpec((H, H), lambda i: (0, 0)),
          pl.BlockSpec((H, H), lambda i: (0, 0)),
          pl.BlockSpec((1, H), lambda i: (0, 0)),
          pl.BlockSpec((H, H), lambda i: (0, 0)),
          pl.BlockSpec((1, H), lambda i: (0, 0)),
      ],
      out_specs=[pl.BlockSpec((BR, H), lambda i: (i, 0)),
                 pl.BlockSpec((BR, H), lambda i: (i, 0))],
      out_shape=[jax.ShapeDtypeStruct((N, H), jnp.float32),
                 jax.ShapeDtypeStruct((N, H), jnp.float32)],
  )(h, m_full, wg1, wg2, b_g, w_4, b_vec)


def _rgcn_dense(s_par, r_par, h_in, w_n, w_l, u=None, v_attr=None):
  fuse = u is not None

  def f(*refs):
    if fuse:
      (s_ref, r_ref, h_ref, wn_ref, wl_ref, u_ref, va_ref, o_ref) = refs
    else:
      (s_ref, r_ref, h_ref, wn_ref, wl_ref, o_ref) = refs
    x = (_dot(s_ref[0] + s_ref[1] + r_ref[0] + r_ref[1], wn_ref[...])
         + _dot(h_ref[...], wl_ref[...]))
    hh = jnp.where(x >= 0, x, x * _RRELU_SLOPE)
    if fuse:
      ub = u_ref[...]
      o_ref[...] = ub * hh + (1.0 - ub) * va_ref[...]
    else:
      o_ref[...] = hh

  spec3 = pl.BlockSpec((NC, BR, H), lambda i: (0, i, 0))
  specb = pl.BlockSpec((BR, H), lambda i: (i, 0))
  specw = pl.BlockSpec((H, H), lambda i: (0, 0))
  in_specs = [spec3, spec3, specb, specw, specw]
  args = [s_par, r_par, h_in, w_n, w_l]
  if fuse:
    in_specs += [specb, specb]
    args += [u, v_attr]
  return pl.pallas_call(
      f, grid=(_G,), in_specs=in_specs,
      out_specs=specb,
      out_shape=jax.ShapeDtypeStruct((N, H), jnp.float32),
  )(*args)


def kernel(rel_src, rel_dst, rel_type, node_id, attr_pair_id, attr_dst,
           dynamic_emb, emb_rel, q, W_m, W_vm, W_g, b_g, W_4, b_vec,
           rgcn_w_neigh_1, rgcn_w_loop_1, rgcn_w_neigh_2, rgcn_w_loop_2):
  h = dynamic_emb
  e = rel_src.shape[0]
  n_chunks = -(-e // (NW * E_CH))
  e_pad = NW * E_CH * n_chunks

  def pad1(x, total, fill):
    return jnp.concatenate(
        [x.astype(jnp.int32), jnp.full((total - x.shape[0],), fill, jnp.int32)])

  src_p = pad1(rel_src, e_pad, 0).reshape(NW, n_chunks, E_CH)
  dst_p = pad1(rel_dst, e_pad, N).reshape(NW, n_chunks, E_CH)
  typ_p = pad1(rel_type, e_pad, 0).reshape(NW, n_chunks, E_CH)
  adst_p = pad1(attr_dst, P_PAD, 0)
  sidx = attr_dst.astype(jnp.int32) * 16 + (attr_pair_id % NA).astype(jnp.int32)
  sidx_p = pad1(sidx, P_PAD, 0)
  zeros = jnp.zeros((ACC_ROWS, H), jnp.float32)

  r_par = _seg_sum_rows(emb_rel, typ_p, dst_p, zeros, n_chunks)
  s1_par = _seg_sum_rows(h, src_p, dst_p, zeros, n_chunks)

  p16 = _prep_p16(h, emb_rel, W_m, W_vm, q)
  m_full = _attr_m(p16.reshape(-1), h, adst_p, sidx_p)
  v_attr, u_gate = _gate_u(h, m_full, W_g[:H], W_g[H:], b_g, W_4, b_vec)

  hh1 = _rgcn_dense(s1_par, r_par, h, rgcn_w_neigh_1, rgcn_w_loop_1)
  s2_par = _seg_sum_rows(hh1, src_p, dst_p, zeros, n_chunks)
  h_new = _rgcn_dense(s2_par, r_par, hh1, rgcn_w_neigh_2,
                      rgcn_w_loop_2, u=u_gate, v_attr=v_attr)
  return (h_new, emb_rel)
